# baseline (speedup 1.0000x reference)
import functools

import jax
import jax.numpy as jnp
from jax import lax
from jax.experimental import pallas as pl
from jax.experimental.pallas import tpu as pltpu
from jax.experimental.pallas import tpu_sc as plsc

_NC = 2
_NS = 16


def _mm(x, W):
    N, D = x.shape
    Dout = W.shape[1]
    R = 1000
    grid = N // R

    def body(x_ref, w_ref, o_ref):
        o_ref[...] = jnp.dot(x_ref[...], w_ref[...],
                             preferred_element_type=jnp.float32)

    return pl.pallas_call(
        body,
        grid=(grid,),
        in_specs=[
            pl.BlockSpec((R, D), lambda i: (i, 0)),
            pl.BlockSpec((D, Dout), lambda i: (0, 0)),
        ],
        out_specs=pl.BlockSpec((R, Dout), lambda i: (i, 0)),
        out_shape=jax.ShapeDtypeStruct((N, Dout), jnp.float32),
    )(x, W)


def _fuse_mm(p, W):
    _, N, D = p.shape
    Dout = W.shape[1]
    R = 1000
    grid = N // R

    def body(p_ref, w_ref, o_ref):
        h = jnp.maximum(p_ref[0] + p_ref[1], 0.0)
        o_ref[...] = jnp.dot(h, w_ref[...],
                             preferred_element_type=jnp.float32)

    return pl.pallas_call(
        body,
        grid=(grid,),
        in_specs=[
            pl.BlockSpec((2, R, D), lambda i: (0, i, 0)),
            pl.BlockSpec((D, Dout), lambda i: (0, 0)),
        ],
        out_specs=pl.BlockSpec((R, Dout), lambda i: (i, 0)),
        out_shape=jax.ShapeDtypeStruct((N, Dout), jnp.float32),
    )(p, W)


def _finish(p):
    _, N, D = p.shape
    R = 1000
    grid = N // R

    def body(p_ref, o_ref):
        z = p_ref[0] + p_ref[1]
        m = jnp.max(z, axis=1, keepdims=True)
        lse = jnp.log(jnp.sum(jnp.exp(z - m), axis=1, keepdims=True)) + m
        o_ref[...] = z - lse

    return pl.pallas_call(
        body,
        grid=(grid,),
        in_specs=[pl.BlockSpec((2, R, D), lambda i: (0, i, 0))],
        out_specs=pl.BlockSpec((R, D), lambda i: (i, 0)),
        out_shape=jax.ShapeDtypeStruct((N, D), jnp.float32),
    )(p)


_C = 80


def _sc_aggregate(m, pk3, steps, zeros, tc_tiling=True):
    N, D = m.shape
    per_w = pk3.shape[0] // (_NC * _NS)
    C = _C
    rows_per_tile = (N // _NS) // 8 * 8
    tail_r0 = _NS * rows_per_tile
    tail_rows = N - tail_r0

    mesh = plsc.VectorSubcoreMesh(core_axis_name="c", subcore_axis_name="s")

    @functools.partial(
        pl.kernel,
        out_type=jax.ShapeDtypeStruct((_NC, N, D), jnp.float32),
        mesh=mesh,
        compiler_params=None if tc_tiling else pltpu.CompilerParams(
            use_tc_tiling_on_sc=False),
        scratch_types=[
            pltpu.VMEM_SHARED((N, D), jnp.float32),
            [pltpu.VMEM((C,), jnp.int32)] * 4,
            [pltpu.VMEM((C,), jnp.int32)] * 4,
            [pltpu.VMEM((C,), jnp.int32)] * 4,
            [pltpu.VMEM((C, D), jnp.float32)] * 4,
            [pltpu.SemaphoreType.DMA] * 4,
            [pltpu.SemaphoreType.DMA] * 4,
            [pltpu.SemaphoreType.DMA] * 4,
        ],
    )
    def body(m_hbm, pk_hbm, zeros_hbm, out_hbm,
             accum, pk_v, src_st, dst_st, rows_v, psem, gsem, ssem):
        c = lax.axis_index("c")
        s = lax.axis_index("s")
        wid = c * _NS + s
        r0 = s * rows_per_tile

        pltpu.sync_copy(zeros_hbm.at[pl.ds(r0, rows_per_tile)],
                        accum.at[pl.ds(r0, rows_per_tile)])

        @pl.when(s == _NS - 1)
        def _():
            pltpu.sync_copy(zeros_hbm.at[pl.ds(tail_r0, tail_rows)],
                            accum.at[pl.ds(tail_r0, tail_rows)])

        plsc.subcore_barrier()

        def pk_off(g):
            return pl.multiple_of(wid * per_w + g * C, 8)

        def pk_start(g, j):
            pltpu.async_copy(pk_hbm.at[pl.ds(pk_off(g), C)], pk_v[j],
                             psem[j])

        def pk_wait(g, j):
            pltpu.make_async_copy(pk_hbm.at[pl.ds(pk_off(g), C)], pk_v[j],
                                  psem[j]).wait()

        def unpack(j):
            for k in range(C // 16):
                v = pk_v[j][pl.ds(k * 16, 16)]
                src_st[j][pl.ds(k * 16, 16)] = v & 0x3FFF
                dst_st[j][pl.ds(k * 16, 16)] = v >> 14

        def gather_start(j):
            pltpu.async_copy(m_hbm.at[src_st[j]], rows_v[j], gsem[j])

        def gather_wait(j):
            pltpu.make_async_copy(m_hbm.at[src_st[j]], rows_v[j],
                                  gsem[j]).wait()

        def scatter_start(j):
            pltpu.async_copy(rows_v[j], accum.at[dst_st[j]], ssem[j],
                             add=True)

        def scatter_wait(j):
            pltpu.make_async_copy(rows_v[j], accum.at[pl.ds(0, C)],
                                  ssem[j]).wait()

        for g in (0, 1, 2, 3):
            pk_start(g, g)
        for g in (0, 1):
            pk_wait(g, g)
            unpack(g)
            gather_start(g)

        def phase(g, j):
            jp = (j + 3) % 4
            j2 = (j + 2) % 4
            g = jnp.int32(g)

            @pl.when(g >= 1)
            def _():
                scatter_wait(jp)

            @pl.when(g + 2 < steps)
            def _():
                pk_wait(g + 2, j2)
                unpack(j2)
                gather_start(j2)

            @pl.when(g + 4 < steps)
            def _():
                pk_start(g + 4, j)

            gather_wait(j)
            scatter_start(j)

        def step(i, carry):
            phase(4 * i, 0)
            phase(4 * i + 1, 1)
            phase(4 * i + 2, 2)
            phase(4 * i + 3, 3)
            return carry

        full = steps // 4
        lax.fori_loop(0, full, step, 0)
        for g in range(full * 4, steps):
            phase(g, g % 4)
        scatter_wait((steps - 1) % 4)
        plsc.subcore_barrier()

        pltpu.sync_copy(accum.at[pl.ds(r0, rows_per_tile)],
                        out_hbm.at[c, pl.ds(r0, rows_per_tile)])

        @pl.when(s == _NS - 1)
        def _():
            pltpu.sync_copy(accum.at[pl.ds(tail_r0, tail_rows)],
                            out_hbm.at[c, pl.ds(tail_r0, tail_rows)])

    return body(m, pk3, zeros)


def kernel(x, edge_index, W_in, W_hid, W_out):
    N = x.shape[0]
    E = edge_index.shape[1]
    NW = _NC * _NS
    src = edge_index[0].astype(jnp.int32)
    dst = edge_index[1].astype(jnp.int32)

    per_w = E // NW
    steps = per_w // _C
    pk3 = jnp.bitwise_or(jnp.left_shift(dst, 14), src)
    z_hid = jnp.zeros((N, W_in.shape[1]), jnp.float32)
    z_out = jnp.zeros((N, W_out.shape[1]), jnp.float32)

    p = _sc_aggregate(_mm(x, W_in), pk3, steps, z_hid)
    p = _sc_aggregate(_fuse_mm(p, W_hid), pk3, steps, z_hid)
    p = _sc_aggregate(_fuse_mm(p, W_out), pk3, steps, z_out, tc_tiling=False)
    return _finish(p)


# speedup vs baseline: 1.0449x; 1.0449x over previous
import functools

import jax
import jax.numpy as jnp
from jax import lax
from jax.experimental import pallas as pl
from jax.experimental.pallas import tpu as pltpu
from jax.experimental.pallas import tpu_sc as plsc

_NC = 2
_NS = 16


def _mm(x, W):
    N, D = x.shape
    Dout = W.shape[1]
    R = 2000
    grid = N // R

    def body(x_ref, w_ref, o_ref):
        o_ref[...] = jnp.dot(x_ref[...], w_ref[...],
                             preferred_element_type=jnp.float32)

    return pl.pallas_call(
        body,
        grid=(grid,),
        in_specs=[
            pl.BlockSpec((R, D), lambda i: (i, 0)),
            pl.BlockSpec((D, Dout), lambda i: (0, 0)),
        ],
        out_specs=pl.BlockSpec((R, Dout), lambda i: (i, 0)),
        out_shape=jax.ShapeDtypeStruct((N, Dout), jnp.float32),
    )(x, W)


def _fuse_mm(p, W):
    _, N, D = p.shape
    Dout = W.shape[1]
    R = 2000
    grid = N // R

    def body(p_ref, w_ref, o_ref):
        h = jnp.maximum(p_ref[0] + p_ref[1], 0.0)
        o_ref[...] = jnp.dot(h, w_ref[...],
                             preferred_element_type=jnp.float32)

    return pl.pallas_call(
        body,
        grid=(grid,),
        in_specs=[
            pl.BlockSpec((2, R, D), lambda i: (0, i, 0)),
            pl.BlockSpec((D, Dout), lambda i: (0, 0)),
        ],
        out_specs=pl.BlockSpec((R, Dout), lambda i: (i, 0)),
        out_shape=jax.ShapeDtypeStruct((N, Dout), jnp.float32),
    )(p, W)


def _finish(p):
    _, N, D = p.shape
    R = 2000
    grid = N // R

    def body(p_ref, o_ref):
        z = p_ref[0] + p_ref[1]
        m = jnp.max(z, axis=1, keepdims=True)
        lse = jnp.log(jnp.sum(jnp.exp(z - m), axis=1, keepdims=True)) + m
        o_ref[...] = z - lse

    return pl.pallas_call(
        body,
        grid=(grid,),
        in_specs=[pl.BlockSpec((2, R, D), lambda i: (0, i, 0))],
        out_specs=pl.BlockSpec((R, D), lambda i: (i, 0)),
        out_shape=jax.ShapeDtypeStruct((N, D), jnp.float32),
    )(p)


_C = 80


def _sc_aggregate(m, pk3, steps, zeros, tc_tiling=True):
    N, D = m.shape
    per_w = pk3.shape[0] // (_NC * _NS)
    C = _C
    rows_per_tile = (N // _NS) // 8 * 8
    tail_r0 = _NS * rows_per_tile
    tail_rows = N - tail_r0

    mesh = plsc.VectorSubcoreMesh(core_axis_name="c", subcore_axis_name="s")

    @functools.partial(
        pl.kernel,
        out_type=jax.ShapeDtypeStruct((_NC, N, D), jnp.float32),
        mesh=mesh,
        compiler_params=None if tc_tiling else pltpu.CompilerParams(
            use_tc_tiling_on_sc=False),
        scratch_types=[
            pltpu.VMEM_SHARED((N, D), jnp.float32),
            [pltpu.VMEM((C,), jnp.int32)] * 4,
            [pltpu.VMEM((C,), jnp.int32)] * 4,
            [pltpu.VMEM((C,), jnp.int32)] * 4,
            [pltpu.VMEM((C, D), jnp.float32)] * 4,
            [pltpu.SemaphoreType.DMA] * 4,
            [pltpu.SemaphoreType.DMA] * 4,
            [pltpu.SemaphoreType.DMA] * 4,
        ],
    )
    def body(m_hbm, pk_hbm, zeros_hbm, out_hbm,
             accum, pk_v, src_st, dst_st, rows_v, psem, gsem, ssem):
        c = lax.axis_index("c")
        s = lax.axis_index("s")
        wid = c * _NS + s
        r0 = s * rows_per_tile

        def pk_off(g):
            return pl.multiple_of(wid * per_w + g * C, 8)

        def pk_start(g, j):
            pltpu.async_copy(pk_hbm.at[pl.ds(pk_off(g), C)], pk_v[j],
                             psem[j])

        def pk_wait(g, j):
            pltpu.make_async_copy(pk_hbm.at[pl.ds(pk_off(g), C)], pk_v[j],
                                  psem[j]).wait()

        def unpack(j):
            for k in range(C // 16):
                v = pk_v[j][pl.ds(k * 16, 16)]
                src_st[j][pl.ds(k * 16, 16)] = v & 0x3FFF
                dst_st[j][pl.ds(k * 16, 16)] = v >> 14

        def gather_start(j):
            pltpu.async_copy(m_hbm.at[src_st[j]], rows_v[j], gsem[j])

        def gather_wait(j):
            pltpu.make_async_copy(m_hbm.at[src_st[j]], rows_v[j],
                                  gsem[j]).wait()

        def scatter_start(j):
            pltpu.async_copy(rows_v[j], accum.at[dst_st[j]], ssem[j],
                             add=True)

        def scatter_wait(j):
            pltpu.make_async_copy(rows_v[j], accum.at[pl.ds(0, C)],
                                  ssem[j]).wait()

        for g in (0, 1, 2, 3):
            pk_start(g, g)
        for g in (0, 1):
            pk_wait(g, g)
            unpack(g)
            gather_start(g)

        pltpu.sync_copy(zeros_hbm.at[pl.ds(r0, rows_per_tile)],
                        accum.at[pl.ds(r0, rows_per_tile)])

        @pl.when(s == _NS - 1)
        def _():
            pltpu.sync_copy(zeros_hbm.at[pl.ds(tail_r0, tail_rows)],
                            accum.at[pl.ds(tail_r0, tail_rows)])

        plsc.subcore_barrier()

        def phase(g, j):
            jp = (j + 3) % 4
            j2 = (j + 2) % 4
            g = jnp.int32(g)

            @pl.when(g >= 1)
            def _():
                scatter_wait(jp)

            @pl.when(g + 2 < steps)
            def _():
                pk_wait(g + 2, j2)
                unpack(j2)
                gather_start(j2)

            @pl.when(g + 4 < steps)
            def _():
                pk_start(g + 4, j)

            gather_wait(j)
            scatter_start(j)

        def step(i, carry):
            phase(4 * i, 0)
            phase(4 * i + 1, 1)
            phase(4 * i + 2, 2)
            phase(4 * i + 3, 3)
            return carry

        full = steps // 4
        lax.fori_loop(0, full, step, 0)
        for g in range(full * 4, steps):
            phase(g, g % 4)
        scatter_wait((steps - 1) % 4)
        plsc.subcore_barrier()

        pltpu.sync_copy(accum.at[pl.ds(r0, rows_per_tile)],
                        out_hbm.at[c, pl.ds(r0, rows_per_tile)])

        @pl.when(s == _NS - 1)
        def _():
            pltpu.sync_copy(accum.at[pl.ds(tail_r0, tail_rows)],
                            out_hbm.at[c, pl.ds(tail_r0, tail_rows)])

    return body(m, pk3, zeros)


def kernel(x, edge_index, W_in, W_hid, W_out):
    N = x.shape[0]
    E = edge_index.shape[1]
    NW = _NC * _NS
    src = edge_index[0].astype(jnp.int32)
    dst = edge_index[1].astype(jnp.int32)

    per_w = E // NW
    steps = per_w // _C
    pk3 = jnp.bitwise_or(jnp.left_shift(dst, 14), src)
    z_hid = jnp.zeros((N, W_in.shape[1]), jnp.float32)
    z_out = jnp.zeros((N, W_out.shape[1]), jnp.float32)

    p = _sc_aggregate(_mm(x, W_in), pk3, steps, z_hid)
    p = _sc_aggregate(_fuse_mm(p, W_hid), pk3, steps, z_hid)
    p = _sc_aggregate(_fuse_mm(p, W_out), pk3, steps, z_out, tc_tiling=False)
    return _finish(p)
